# baseline (speedup 1.0000x reference)
import functools

import jax
import jax.numpy as jnp
from jax import lax
from jax.experimental import pallas as pl
from jax.experimental.pallas import tpu as pltpu
from jax.experimental.pallas import tpu_sc as plsc

N_EXPERTS = 64
TOP_K = 8
TOK_BLOCK_TC = 512


def _tc_logits_body(x_ref, w_ref, out_ref):
    out_ref[...] = lax.dot_general(
        x_ref[...],
        w_ref[...],
        dimension_numbers=(((1,), (1,)), ((), ())),
        preferred_element_type=jnp.float32,
    )


def _tc_logits(x, w):
    t, h = x.shape
    return pl.pallas_call(
        _tc_logits_body,
        grid=(t // TOK_BLOCK_TC,),
        in_specs=[
            pl.BlockSpec((TOK_BLOCK_TC, h), lambda i: (i, 0)),
            pl.BlockSpec((N_EXPERTS, h), lambda i: (0, 0)),
        ],
        out_specs=pl.BlockSpec((TOK_BLOCK_TC, N_EXPERTS), lambda i: (i, 0)),
        out_shape=jax.ShapeDtypeStruct((t, N_EXPERTS), jnp.float32),
    )(x, w)


def _sc_topk(logits):
    t = logits.shape[0] // N_EXPERTS
    info = plsc.get_sparse_core_info()
    nc, ns, lanes = info.num_cores, info.num_subcores, info.num_lanes
    nw = nc * ns
    per_w = t // nw
    n_blocks = per_w // lanes
    mesh = plsc.VectorSubcoreMesh(core_axis_name="c", subcore_axis_name="s")

    @functools.partial(
        pl.kernel,
        mesh=mesh,
        out_type=[
            jax.ShapeDtypeStruct((t * TOP_K,), jnp.float32),
            jax.ShapeDtypeStruct((t * TOP_K,), jnp.int32),
        ],
        scratch_types=[
            pltpu.VMEM((lanes * N_EXPERTS,), jnp.float32),
            pltpu.VMEM((per_w * TOP_K,), jnp.float32),
            pltpu.VMEM((per_w * TOP_K,), jnp.int32),
        ],
        compiler_params=pltpu.CompilerParams(needs_layout_passes=False),
    )
    def k(logits_hbm, outw_hbm, outi_hbm, lblk, wv, iv):
        wid = lax.axis_index("s") * nc + lax.axis_index("c")
        base = wid * per_w
        rows = lax.iota(jnp.int32, lanes)
        rows_scaled = rows * N_EXPERTS
        neg = jnp.full((lanes,), -jnp.inf, jnp.float32)

        def block(b, carry):
            tok0 = base + b * lanes
            pltpu.sync_copy(
                logits_hbm.at[pl.ds(tok0 * N_EXPERTS, lanes * N_EXPERTS)], lblk
            )
            tvals = [neg] * TOP_K
            tidx = [jnp.zeros((lanes,), jnp.int32)] * TOP_K
            for e in range(N_EXPERTS):
                x = plsc.load_gather(lblk, [rows_scaled + e])
                xi = jnp.full((lanes,), e, jnp.int32)
                for j in range(TOP_K):
                    c = x > tvals[j]
                    hi = jnp.maximum(tvals[j], x)
                    lo = jnp.minimum(tvals[j], x)
                    ii = jnp.where(c, xi, tidx[j])
                    xi = jnp.where(c, tidx[j], xi)
                    tvals[j] = hi
                    x = lo
                    tidx[j] = ii
            exps = [jnp.exp(tvals[j] - tvals[0]) for j in range(TOP_K)]
            s = exps[0]
            for j in range(1, TOP_K):
                s = s + exps[j]
            r = 1.0 / s
            loc = (b * lanes + rows) * TOP_K
            for j in range(TOP_K):
                plsc.store_scatter(wv, [loc + j], exps[j] * r)
                plsc.store_scatter(iv, [loc + j], tidx[j])
            return carry

        lax.fori_loop(0, n_blocks, block, 0)
        pltpu.sync_copy(wv, outw_hbm.at[pl.ds(base * TOP_K, per_w * TOP_K)])
        pltpu.sync_copy(iv, outi_hbm.at[pl.ds(base * TOP_K, per_w * TOP_K)])

    return k(logits)


def kernel(hidden_states, weight):
    b, s, h = hidden_states.shape
    x = hidden_states.reshape(-1, h)
    t = x.shape[0]
    logits = _tc_logits(x, weight)
    w_flat, i_flat = _sc_topk(logits.reshape(-1))
    return w_flat.reshape(t, TOP_K), i_flat.reshape(t, TOP_K)
